# baseline (speedup 1.0000x reference)
import functools

import jax
import jax.numpy as jnp
from jax import lax
from jax.experimental import pallas as pl
from jax.experimental.pallas import tpu as pltpu
from jax.experimental.pallas import tpu_sc as plsc

N_NODES = 10000
D = 128
N_CLASSES = 40

NC = 2
NS = 16
NW = NC * NS
K = 128
NE_PAD = 327680
EPW = NE_PAD // NW
NCH = EPW // K
NBUF = 4
ACC_ROWS = 10240
ZPT = ACC_ROWS // NS
RPT = 624


def _sc_agg_body(with_deg, table, src2d, dst2d, z2d, z1d, *rest):
    if with_deg:
        (out, degout, acc, deg_s, sring, dbuf, rows, onesv, degb,
         gsem, isem, ssem, dsem) = rest
    else:
        out, acc, sring, dbuf, rows, gsem, isem, ssem, dsem = rest
    c = lax.axis_index("c")
    s = lax.axis_index("s")
    w = c * NS + s
    ib = w * NCH

    pltpu.sync_copy(z2d, acc.at[pl.ds(s * ZPT, ZPT)])
    pltpu.sync_copy(dst2d.at[pl.ds(ib, NCH)], dbuf)
    for j in range(3):
        pltpu.sync_copy(src2d.at[ib + j], sring.at[j])
    if with_deg:
        pltpu.sync_copy(z1d, deg_s.at[pl.ds(s * ZPT, ZPT)])
        for j in range(K // 16):
            onesv[pl.ds(j * 16, 16)] = jnp.ones((16,), jnp.float32)
    plsc.subcore_barrier()

    pltpu.async_copy(src2d.at[ib + 3], sring.at[3], isem)
    pltpu.async_copy(table.at[sring.at[0]], rows.at[0], gsem)

    def step(i, carry):
        b = lax.rem(i, 2)
        pltpu.make_async_copy(table.at[sring.at[0]], rows.at[b], gsem).wait()

        @pl.when(i + 3 < NCH)
        def _():
            pltpu.make_async_copy(src2d.at[ib], sring.at[0], isem).wait()

        @pl.when(i >= 1)
        def _():
            pltpu.make_async_copy(rows.at[b], acc.at[dbuf.at[0]],
                                  ssem).wait()

        @pl.when(i + 1 < NCH)
        def _():
            pltpu.async_copy(table.at[sring.at[lax.rem(i + 1, 4)]],
                             rows.at[1 - b], gsem)

        @pl.when(i + 4 < NCH)
        def _():
            pltpu.async_copy(src2d.at[ib + i + 4],
                             sring.at[lax.rem(i, 4)], isem)

        pltpu.async_copy(rows.at[b], acc.at[dbuf.at[i]], ssem, add=True)
        if with_deg:
            pltpu.async_copy(onesv, deg_s.at[dbuf.at[i]], dsem, add=True)

            @pl.when(i >= 8)
            def _():
                pltpu.make_async_copy(onesv, deg_s.at[dbuf.at[0]],
                                      dsem).wait()
        return carry

    lax.fori_loop(0, NCH, step, 0)
    pltpu.make_async_copy(rows.at[0], acc.at[dbuf.at[0]], ssem).wait()
    if with_deg:
        for _ in range(8):
            pltpu.make_async_copy(onesv, deg_s.at[dbuf.at[0]], dsem).wait()
    plsc.subcore_barrier()

    pltpu.sync_copy(acc.at[pl.ds(s * RPT, RPT)], out.at[c, pl.ds(s * RPT, RPT)])
    if with_deg:
        pltpu.sync_copy(deg_s.at[pl.ds(s * RPT, RPT)], degb.at[pl.ds(0, RPT)])
        pltpu.sync_copy(degb.at[pl.ds(0, RPT)],
                        degout.at[pl.ds(c * N_NODES + s * RPT, RPT)])

    @pl.when(s == 0)
    def _():
        tail = N_NODES - NS * RPT
        pltpu.sync_copy(acc.at[pl.ds(NS * RPT, tail)],
                        out.at[c, pl.ds(NS * RPT, tail)])
        if with_deg:
            pltpu.sync_copy(deg_s.at[pl.ds(NS * RPT, tail)],
                            degb.at[pl.ds(0, tail)])
            pltpu.sync_copy(degb.at[pl.ds(0, tail)],
                            degout.at[pl.ds(c * N_NODES + NS * RPT, tail)])


def _make_sc_agg(with_deg):
    mesh = plsc.VectorSubcoreMesh(core_axis_name="c", subcore_axis_name="s")
    common = [
        pltpu.VMEM_SHARED((ACC_ROWS, D), jnp.float32),
    ]
    bufs = [
        pltpu.VMEM((4, K), jnp.int32),
        pltpu.VMEM((NCH, K), jnp.int32),
        pltpu.VMEM((2, K, D), jnp.float32),
    ]
    sems = [pltpu.SemaphoreType.DMA] * 4
    if with_deg:
        out_type = (
            jax.ShapeDtypeStruct((NC, N_NODES, D), jnp.float32),
            jax.ShapeDtypeStruct((NC * N_NODES,), jnp.float32),
        )
        scratch = common + [pltpu.VMEM_SHARED((ACC_ROWS,), jnp.float32)] \
            + bufs + [pltpu.VMEM((K,), jnp.float32),
                      pltpu.VMEM((ZPT,), jnp.float32)] + sems
    else:
        out_type = jax.ShapeDtypeStruct((NC, N_NODES, D), jnp.float32)
        scratch = common + bufs + sems
    return pl.kernel(
        functools.partial(_sc_agg_body, with_deg),
        out_type=out_type,
        mesh=mesh,
        scratch_types=scratch,
    )


BR = 1000


def _mean_agg(sp_ref, degT_ref):
    ssum = sp_ref[0] + sp_ref[1]
    deg = jnp.sum(degT_ref[...], axis=1, keepdims=True)
    invd = 1.0 / jnp.clip(deg, 1.0, None)
    return ssum * invd


def _dense1_body(x_ref, sp_ref, degT_ref, Ws_ref, Wn_ref, b_ref, o_ref):
    agg = _mean_agg(sp_ref, degT_ref)
    h = jnp.dot(x_ref[...], Ws_ref[...], preferred_element_type=jnp.float32)
    h = h + jnp.dot(agg, Wn_ref[...], preferred_element_type=jnp.float32)
    h = h + b_ref[...][None, :]
    o_ref[...] = jnp.maximum(h, 0.0)


def _dense2_body(x_ref, sp_ref, degT_ref, Ws_ref, Wn_ref, b_ref, Wo_ref,
                 bo_ref, o_ref):
    agg = _mean_agg(sp_ref, degT_ref)
    h = jnp.dot(x_ref[...], Ws_ref[...], preferred_element_type=jnp.float32)
    h = h + jnp.dot(agg, Wn_ref[...], preferred_element_type=jnp.float32)
    h = jnp.maximum(h + b_ref[...][None, :], 0.0)
    o_ref[...] = (
        jnp.dot(h, Wo_ref[...], preferred_element_type=jnp.float32)
        + bo_ref[...][None, :]
    )


_W_SPEC = pl.BlockSpec((D, D), lambda i: (0, 0))
_B_SPEC = pl.BlockSpec((D,), lambda i: (0,))
_X_SPEC = pl.BlockSpec((BR, D), lambda i: (i, 0))
_SP_SPEC = pl.BlockSpec((NC, BR, D), lambda i: (0, i, 0))
_DEG_SPEC = pl.BlockSpec((BR, NC), lambda i: (i, 0))

_dense1 = pl.pallas_call(
    _dense1_body,
    grid=(N_NODES // BR,),
    in_specs=[_X_SPEC, _SP_SPEC, _DEG_SPEC, _W_SPEC, _W_SPEC, _B_SPEC],
    out_specs=_X_SPEC,
    out_shape=jax.ShapeDtypeStruct((N_NODES, D), jnp.float32),
)

_dense2 = pl.pallas_call(
    _dense2_body,
    grid=(N_NODES // BR,),
    in_specs=[_X_SPEC, _SP_SPEC, _DEG_SPEC, _W_SPEC, _W_SPEC, _B_SPEC,
              _W_SPEC, _B_SPEC],
    out_specs=_X_SPEC,
    out_shape=jax.ShapeDtypeStruct((N_NODES, D), jnp.float32),
)

_sc_agg_deg = _make_sc_agg(True)
_sc_agg = _make_sc_agg(False)


def kernel(features, edge_index, W_self1, W_neigh1, b1, W_self2, W_neigh2,
           b2, W_out, b_out):
    pad_n = NE_PAD - edge_index.shape[1]
    ar = jnp.arange(pad_n, dtype=jnp.int32)
    pad_src = (ar * 13) % N_NODES
    pad_dst = N_NODES + ar % (ACC_ROWS - N_NODES)
    src2d = jnp.concatenate([edge_index[0], pad_src]).reshape(NW * NCH, K)
    dst2d = jnp.concatenate([edge_index[1], pad_dst]).reshape(NW * NCH, K)
    z2d = jnp.zeros((ZPT, D), jnp.float32)
    z1d = jnp.zeros((ZPT,), jnp.float32)

    sp1, deg_flat = _sc_agg_deg(features, src2d, dst2d, z2d, z1d)
    degT = deg_flat.reshape(NC, N_NODES).T
    h1 = _dense1(features, sp1, degT, W_self1, W_neigh1, b1)
    sp2 = _sc_agg(h1, src2d, dst2d, z2d, z1d)

    Wo_p = jnp.zeros((D, D), jnp.float32).at[:, :N_CLASSES].set(W_out)
    bo_p = jnp.zeros((D,), jnp.float32).at[:N_CLASSES].set(b_out)
    out_p = _dense2(h1, sp2, degT, W_self2, W_neigh2, b2, Wo_p, bo_p)
    return out_p[:, :N_CLASSES]


# speedup vs baseline: 1.2321x; 1.2321x over previous
import functools

import jax
import jax.numpy as jnp
from jax import lax
from jax.experimental import pallas as pl
from jax.experimental.pallas import tpu as pltpu
from jax.experimental.pallas import tpu_sc as plsc

N_NODES = 10000
D = 128
N_CLASSES = 40

NC = 2
NS = 16
NW = NC * NS
K = 128
NE_PAD = 327680
EPW = NE_PAD // NW
NCH = EPW // K
NBUF = 4
ACC_ROWS = 10016
DEG_ROWS = 10240
RPT = 624


def _sc_agg_body(with_deg, table, src2d, dst2d, z2d, z1d, *rest):
    if with_deg:
        (out, degout, acc, deg_s, sring, dring, rows, degb,
         gsem, isem, vsem, ssem, dsem) = rest
    else:
        out, acc, sring, dring, rows, gsem, isem, vsem, ssem, dsem = rest
    c = lax.axis_index("c")
    s = lax.axis_index("s")
    w = c * NS + s
    ib = w * NCH

    tail = N_NODES - NS * RPT
    pltpu.sync_copy(z2d, acc.at[pl.ds(s * RPT, RPT)])
    for j in range(3):
        pltpu.sync_copy(src2d.at[ib + j], sring.at[j])
    for j in range(2):
        pltpu.sync_copy(dst2d.at[ib + j], dring.at[j])
    if with_deg:
        pltpu.sync_copy(z1d, deg_s.at[pl.ds(s * (DEG_ROWS // NS),
                                            DEG_ROWS // NS)])
        for j in range(K // 16):
            degb[pl.ds(j * 16, 16)] = jnp.ones((16,), jnp.float32)

    @pl.when(s == 0)
    def _():
        pltpu.sync_copy(z2d.at[pl.ds(0, tail)], acc.at[pl.ds(NS * RPT, tail)])
    plsc.subcore_barrier()

    pltpu.async_copy(table.at[sring.at[0]], rows.at[0], gsem)
    pltpu.async_copy(table.at[sring.at[1]], rows.at[1], gsem)

    def step(i, carry):
        b = lax.rem(i, 3)
        pltpu.make_async_copy(table.at[sring.at[0]], rows.at[b], gsem).wait()

        @pl.when(i >= 1)
        def _():
            pltpu.make_async_copy(rows.at[0], acc.at[dring.at[0]],
                                  ssem).wait()
            if with_deg:
                pltpu.make_async_copy(degb, deg_s.at[dring.at[0]],
                                      dsem).wait()

        @pl.when((i >= 1) & (i + 2 < NCH))
        def _():
            pltpu.make_async_copy(src2d.at[ib], sring.at[0], isem).wait()

        @pl.when((i >= 1) & (i + 1 < NCH))
        def _():
            pltpu.make_async_copy(dst2d.at[ib], dring.at[0], vsem).wait()

        @pl.when(i + 2 < NCH)
        def _():
            pltpu.async_copy(table.at[sring.at[lax.rem(i + 2, 3)]],
                             rows.at[lax.rem(i + 2, 3)], gsem)
            pltpu.async_copy(dst2d.at[ib + i + 2],
                             dring.at[lax.rem(i + 2, 3)], vsem)

        @pl.when(i + 3 < NCH)
        def _():
            pltpu.async_copy(src2d.at[ib + i + 3],
                             sring.at[lax.rem(i, 3)], isem)

        pltpu.async_copy(rows.at[b], acc.at[dring.at[lax.rem(i, 3)]],
                         ssem, add=True)
        if with_deg:
            pltpu.async_copy(degb, deg_s.at[dring.at[lax.rem(i, 3)]],
                             dsem, add=True)
        return carry

    lax.fori_loop(0, NCH, step, 0)
    pltpu.make_async_copy(rows.at[0], acc.at[dring.at[0]], ssem).wait()
    if with_deg:
        pltpu.make_async_copy(degb, deg_s.at[dring.at[0]], dsem).wait()
    plsc.subcore_barrier()

    pltpu.sync_copy(acc.at[pl.ds(s * RPT, RPT)], out.at[c, pl.ds(s * RPT, RPT)])
    if with_deg:
        for j, sz in ((0, K), (1, K), (2, K), (3, K), (4, RPT - 4 * K)):
            pltpu.sync_copy(deg_s.at[pl.ds(s * RPT + j * K, sz)],
                            degb.at[pl.ds(0, sz)])
            pltpu.sync_copy(degb.at[pl.ds(0, sz)],
                            degout.at[pl.ds(c * N_NODES + s * RPT + j * K,
                                            sz)])

    @pl.when(s == 0)
    def _():
        pltpu.sync_copy(acc.at[pl.ds(NS * RPT, tail)],
                        out.at[c, pl.ds(NS * RPT, tail)])
        if with_deg:
            pltpu.sync_copy(deg_s.at[pl.ds(NS * RPT, tail)],
                            degb.at[pl.ds(0, tail)])
            pltpu.sync_copy(degb.at[pl.ds(0, tail)],
                            degout.at[pl.ds(c * N_NODES + NS * RPT, tail)])


def _make_sc_agg(with_deg):
    mesh = plsc.VectorSubcoreMesh(core_axis_name="c", subcore_axis_name="s")
    common = [
        pltpu.VMEM_SHARED((ACC_ROWS, D), jnp.float32),
    ]
    bufs = [
        pltpu.VMEM((3, K), jnp.int32),
        pltpu.VMEM((3, K), jnp.int32),
        pltpu.VMEM((3, K, D), jnp.float32),
    ]
    sems = [pltpu.SemaphoreType.DMA] * 5
    if with_deg:
        out_type = (
            jax.ShapeDtypeStruct((NC, N_NODES, D), jnp.float32),
            jax.ShapeDtypeStruct((NC * N_NODES,), jnp.float32),
        )
        scratch = common + [pltpu.VMEM_SHARED((DEG_ROWS,), jnp.float32)] \
            + bufs + [pltpu.VMEM((K,), jnp.float32)] + sems
    else:
        out_type = jax.ShapeDtypeStruct((NC, N_NODES, D), jnp.float32)
        scratch = common + bufs + sems
    return pl.kernel(
        functools.partial(_sc_agg_body, with_deg),
        out_type=out_type,
        mesh=mesh,
        scratch_types=scratch,
    )


BR = 1000


def _mean_agg(sp_ref, degT_ref):
    ssum = sp_ref[0] + sp_ref[1]
    deg = jnp.sum(degT_ref[...], axis=1, keepdims=True)
    invd = 1.0 / jnp.clip(deg, 1.0, None)
    return ssum * invd


def _dense1_body(x_ref, sp_ref, degT_ref, Ws_ref, Wn_ref, b_ref, o_ref):
    agg = _mean_agg(sp_ref, degT_ref)
    h = jnp.dot(x_ref[...], Ws_ref[...], preferred_element_type=jnp.float32)
    h = h + jnp.dot(agg, Wn_ref[...], preferred_element_type=jnp.float32)
    h = h + b_ref[...][None, :]
    o_ref[...] = jnp.maximum(h, 0.0)


def _dense2_body(x_ref, sp_ref, degT_ref, Ws_ref, Wn_ref, b_ref, Wo_ref,
                 bo_ref, o_ref):
    agg = _mean_agg(sp_ref, degT_ref)
    h = jnp.dot(x_ref[...], Ws_ref[...], preferred_element_type=jnp.float32)
    h = h + jnp.dot(agg, Wn_ref[...], preferred_element_type=jnp.float32)
    h = jnp.maximum(h + b_ref[...][None, :], 0.0)
    o_ref[...] = (
        jnp.dot(h, Wo_ref[...], preferred_element_type=jnp.float32)
        + bo_ref[...][None, :]
    )


_W_SPEC = pl.BlockSpec((D, D), lambda i: (0, 0))
_B_SPEC = pl.BlockSpec((D,), lambda i: (0,))
_X_SPEC = pl.BlockSpec((BR, D), lambda i: (i, 0))
_SP_SPEC = pl.BlockSpec((NC, BR, D), lambda i: (0, i, 0))
_DEG_SPEC = pl.BlockSpec((BR, NC), lambda i: (i, 0))

_dense1 = pl.pallas_call(
    _dense1_body,
    grid=(N_NODES // BR,),
    in_specs=[_X_SPEC, _SP_SPEC, _DEG_SPEC, _W_SPEC, _W_SPEC, _B_SPEC],
    out_specs=_X_SPEC,
    out_shape=jax.ShapeDtypeStruct((N_NODES, D), jnp.float32),
)

_dense2 = pl.pallas_call(
    _dense2_body,
    grid=(N_NODES // BR,),
    in_specs=[_X_SPEC, _SP_SPEC, _DEG_SPEC, _W_SPEC, _W_SPEC, _B_SPEC,
              _W_SPEC, _B_SPEC],
    out_specs=_X_SPEC,
    out_shape=jax.ShapeDtypeStruct((N_NODES, D), jnp.float32),
)

_sc_agg_deg = _make_sc_agg(True)
_sc_agg = _make_sc_agg(False)


def kernel(features, edge_index, W_self1, W_neigh1, b1, W_self2, W_neigh2,
           b2, W_out, b_out):
    pad_n = NE_PAD - edge_index.shape[1]
    ar = jnp.arange(pad_n, dtype=jnp.int32)
    pad_src = (ar * 13) % N_NODES
    pad_dst = N_NODES + ar % (ACC_ROWS - N_NODES)
    src2d = jnp.concatenate([edge_index[0], pad_src]).reshape(NW * NCH, K)
    dst2d = jnp.concatenate([edge_index[1], pad_dst]).reshape(NW * NCH, K)
    z2d = jnp.zeros((RPT, D), jnp.float32)
    z1d = jnp.zeros((DEG_ROWS // NS,), jnp.float32)

    sp1, deg_flat = _sc_agg_deg(features, src2d, dst2d, z2d, z1d)
    degT = deg_flat.reshape(NC, N_NODES).T
    h1 = _dense1(features, sp1, degT, W_self1, W_neigh1, b1)
    sp2 = _sc_agg(h1, src2d, dst2d, z2d, z1d)

    Wo_p = jnp.zeros((D, D), jnp.float32).at[:, :N_CLASSES].set(W_out)
    bo_p = jnp.zeros((D,), jnp.float32).at[:N_CLASSES].set(b_out)
    out_p = _dense2(h1, sp2, degT, W_self2, W_neigh2, b2, Wo_p, bo_p)
    return out_p[:, :N_CLASSES]


# speedup vs baseline: 1.2448x; 1.0103x over previous
import functools

import jax
import jax.numpy as jnp
from jax import lax
from jax.experimental import pallas as pl
from jax.experimental.pallas import tpu as pltpu
from jax.experimental.pallas import tpu_sc as plsc

N_NODES = 10000
D = 128
N_CLASSES = 40

NC = 2
NS = 16
NW = NC * NS
K = 96
NE_PAD = 322560
EPW = NE_PAD // NW
NCH = EPW // K
NBUF = 4
ACC_ROWS = 10016
DEG_ROWS = 10240
RPT = 624


def _sc_agg_body(with_deg, table, src2d, dst2d, z2d, z1d, *rest):
    if with_deg:
        (out, degout, acc, deg_s, sring, dring, rows, degb,
         gsem, isem, vsem, ssem, dsem) = rest
    else:
        out, acc, sring, dring, rows, gsem, isem, vsem, ssem, dsem = rest
    c = lax.axis_index("c")
    s = lax.axis_index("s")
    w = c * NS + s
    ib = w * NCH

    tail = N_NODES - NS * RPT
    pltpu.sync_copy(z2d, acc.at[pl.ds(s * RPT, RPT)])
    for j in range(4):
        pltpu.sync_copy(src2d.at[ib + j], sring.at[j])
    for j in range(3):
        pltpu.sync_copy(dst2d.at[ib + j], dring.at[j])
    if with_deg:
        pltpu.sync_copy(z1d, deg_s.at[pl.ds(s * (DEG_ROWS // NS),
                                            DEG_ROWS // NS)])
        for j in range(K // 16):
            degb[pl.ds(j * 16, 16)] = jnp.ones((16,), jnp.float32)

    @pl.when(s == 0)
    def _():
        pltpu.sync_copy(z2d.at[pl.ds(0, tail)], acc.at[pl.ds(NS * RPT, tail)])
    plsc.subcore_barrier()

    for j in range(3):
        pltpu.async_copy(table.at[sring.at[j]], rows.at[j], gsem)

    def step(i, carry):
        b = lax.rem(i, 4)
        pltpu.make_async_copy(table.at[sring.at[0]], rows.at[b], gsem).wait()

        @pl.when(i >= 1)
        def _():
            pltpu.make_async_copy(rows.at[0], acc.at[dring.at[0]],
                                  ssem).wait()
            if with_deg:
                pltpu.make_async_copy(degb, deg_s.at[dring.at[0]],
                                      dsem).wait()

        @pl.when((i >= 1) & (i + 3 < NCH))
        def _():
            pltpu.make_async_copy(src2d.at[ib], sring.at[0], isem).wait()

        @pl.when((i >= 1) & (i + 2 < NCH))
        def _():
            pltpu.make_async_copy(dst2d.at[ib], dring.at[0], vsem).wait()

        @pl.when(i + 3 < NCH)
        def _():
            pltpu.async_copy(table.at[sring.at[lax.rem(i + 3, 4)]],
                             rows.at[lax.rem(i + 3, 4)], gsem)
            pltpu.async_copy(dst2d.at[ib + i + 3],
                             dring.at[lax.rem(i + 3, 4)], vsem)

        @pl.when(i + 4 < NCH)
        def _():
            pltpu.async_copy(src2d.at[ib + i + 4],
                             sring.at[lax.rem(i, 4)], isem)

        pltpu.async_copy(rows.at[b], acc.at[dring.at[lax.rem(i, 4)]],
                         ssem, add=True)
        if with_deg:
            pltpu.async_copy(degb, deg_s.at[dring.at[lax.rem(i, 4)]],
                             dsem, add=True)
        return carry

    lax.fori_loop(0, NCH, step, 0)
    pltpu.make_async_copy(rows.at[0], acc.at[dring.at[0]], ssem).wait()
    if with_deg:
        pltpu.make_async_copy(degb, deg_s.at[dring.at[0]], dsem).wait()
    plsc.subcore_barrier()

    pltpu.sync_copy(acc.at[pl.ds(s * RPT, RPT)], out.at[c, pl.ds(s * RPT, RPT)])
    if with_deg:
        for j, sz in tuple((j, K) for j in range(6)) + ((6, RPT - 6 * K),):
            pltpu.sync_copy(deg_s.at[pl.ds(s * RPT + j * K, sz)],
                            degb.at[pl.ds(0, sz)])
            pltpu.sync_copy(degb.at[pl.ds(0, sz)],
                            degout.at[pl.ds(c * N_NODES + s * RPT + j * K,
                                            sz)])

    @pl.when(s == 0)
    def _():
        pltpu.sync_copy(acc.at[pl.ds(NS * RPT, tail)],
                        out.at[c, pl.ds(NS * RPT, tail)])
        if with_deg:
            pltpu.sync_copy(deg_s.at[pl.ds(NS * RPT, tail)],
                            degb.at[pl.ds(0, tail)])
            pltpu.sync_copy(degb.at[pl.ds(0, tail)],
                            degout.at[pl.ds(c * N_NODES + NS * RPT, tail)])


def _make_sc_agg(with_deg):
    mesh = plsc.VectorSubcoreMesh(core_axis_name="c", subcore_axis_name="s")
    common = [
        pltpu.VMEM_SHARED((ACC_ROWS, D), jnp.float32),
    ]
    bufs = [
        pltpu.VMEM((4, K), jnp.int32),
        pltpu.VMEM((4, K), jnp.int32),
        pltpu.VMEM((4, K, D), jnp.float32),
    ]
    sems = [pltpu.SemaphoreType.DMA] * 5
    if with_deg:
        out_type = (
            jax.ShapeDtypeStruct((NC, N_NODES, D), jnp.float32),
            jax.ShapeDtypeStruct((NC * N_NODES,), jnp.float32),
        )
        scratch = common + [pltpu.VMEM_SHARED((DEG_ROWS,), jnp.float32)] \
            + bufs + [pltpu.VMEM((K,), jnp.float32)] + sems
    else:
        out_type = jax.ShapeDtypeStruct((NC, N_NODES, D), jnp.float32)
        scratch = common + bufs + sems
    return pl.kernel(
        functools.partial(_sc_agg_body, with_deg),
        out_type=out_type,
        mesh=mesh,
        scratch_types=scratch,
    )


BR = 1000


def _mean_agg(sp_ref, degT_ref):
    ssum = sp_ref[0] + sp_ref[1]
    deg = jnp.sum(degT_ref[...], axis=1, keepdims=True)
    invd = 1.0 / jnp.clip(deg, 1.0, None)
    return ssum * invd


def _dense1_body(x_ref, sp_ref, degT_ref, Ws_ref, Wn_ref, b_ref, o_ref):
    agg = _mean_agg(sp_ref, degT_ref)
    h = jnp.dot(x_ref[...], Ws_ref[...], preferred_element_type=jnp.float32)
    h = h + jnp.dot(agg, Wn_ref[...], preferred_element_type=jnp.float32)
    h = h + b_ref[...][None, :]
    o_ref[...] = jnp.maximum(h, 0.0)


def _dense2_body(x_ref, sp_ref, degT_ref, Ws_ref, Wn_ref, b_ref, Wo_ref,
                 bo_ref, o_ref):
    agg = _mean_agg(sp_ref, degT_ref)
    h = jnp.dot(x_ref[...], Ws_ref[...], preferred_element_type=jnp.float32)
    h = h + jnp.dot(agg, Wn_ref[...], preferred_element_type=jnp.float32)
    h = jnp.maximum(h + b_ref[...][None, :], 0.0)
    o_ref[...] = (
        jnp.dot(h, Wo_ref[...], preferred_element_type=jnp.float32)
        + bo_ref[...][None, :]
    )


_W_SPEC = pl.BlockSpec((D, D), lambda i: (0, 0))
_B_SPEC = pl.BlockSpec((D,), lambda i: (0,))
_X_SPEC = pl.BlockSpec((BR, D), lambda i: (i, 0))
_SP_SPEC = pl.BlockSpec((NC, BR, D), lambda i: (0, i, 0))
_DEG_SPEC = pl.BlockSpec((BR, NC), lambda i: (i, 0))

_dense1 = pl.pallas_call(
    _dense1_body,
    grid=(N_NODES // BR,),
    in_specs=[_X_SPEC, _SP_SPEC, _DEG_SPEC, _W_SPEC, _W_SPEC, _B_SPEC],
    out_specs=_X_SPEC,
    out_shape=jax.ShapeDtypeStruct((N_NODES, D), jnp.float32),
)

_dense2 = pl.pallas_call(
    _dense2_body,
    grid=(N_NODES // BR,),
    in_specs=[_X_SPEC, _SP_SPEC, _DEG_SPEC, _W_SPEC, _W_SPEC, _B_SPEC,
              _W_SPEC, _B_SPEC],
    out_specs=_X_SPEC,
    out_shape=jax.ShapeDtypeStruct((N_NODES, D), jnp.float32),
)

_sc_agg_deg = _make_sc_agg(True)
_sc_agg = _make_sc_agg(False)


def kernel(features, edge_index, W_self1, W_neigh1, b1, W_self2, W_neigh2,
           b2, W_out, b_out):
    pad_n = NE_PAD - edge_index.shape[1]
    ar = jnp.arange(pad_n, dtype=jnp.int32)
    pad_src = (ar * 13) % N_NODES
    pad_dst = N_NODES + ar % (ACC_ROWS - N_NODES)
    src2d = jnp.concatenate([edge_index[0], pad_src]).reshape(NW * NCH, K)
    dst2d = jnp.concatenate([edge_index[1], pad_dst]).reshape(NW * NCH, K)
    z2d = jnp.zeros((RPT, D), jnp.float32)
    z1d = jnp.zeros((DEG_ROWS // NS,), jnp.float32)

    sp1, deg_flat = _sc_agg_deg(features, src2d, dst2d, z2d, z1d)
    degT = deg_flat.reshape(NC, N_NODES).T
    h1 = _dense1(features, sp1, degT, W_self1, W_neigh1, b1)
    sp2 = _sc_agg(h1, src2d, dst2d, z2d, z1d)

    Wo_p = jnp.zeros((D, D), jnp.float32).at[:, :N_CLASSES].set(W_out)
    bo_p = jnp.zeros((D,), jnp.float32).at[:N_CLASSES].set(b_out)
    out_p = _dense2(h1, sp2, degT, W_self2, W_neigh2, b2, Wo_p, bo_p)
    return out_p[:, :N_CLASSES]


# speedup vs baseline: 1.2637x; 1.0152x over previous
import functools

import jax
import jax.numpy as jnp
from jax import lax
from jax.experimental import pallas as pl
from jax.experimental.pallas import tpu as pltpu
from jax.experimental.pallas import tpu_sc as plsc

N_NODES = 10000
D = 128
N_CLASSES = 40

NC = 2
NS = 16
NW = NC * NS
K = 80
N_EDGES = 320000
EPW = N_EDGES // NW
NCH = EPW // K
ACC_ROWS = N_NODES
DEG_ROWS = 10240
RPT = 624


def _sc_agg_body(with_deg, table, src1d, dst1d, z2d, z1d, *rest):
    if with_deg:
        (out, degout, acc, deg_s, sring, dring, rows, degb,
         gsem, isem, vsem, ssem, dsem) = rest
    else:
        out, acc, sring, dring, rows, gsem, isem, vsem, ssem, dsem = rest
    c = lax.axis_index("c")
    s = lax.axis_index("s")
    w = c * NS + s
    ib = w * EPW

    tail = N_NODES - NS * RPT
    pltpu.sync_copy(z2d, acc.at[pl.ds(s * RPT, RPT)])
    for j in range(4):
        pltpu.sync_copy(src1d.at[pl.ds(ib + j * K, K)], sring.at[j])
    for j in range(3):
        pltpu.sync_copy(dst1d.at[pl.ds(ib + j * K, K)], dring.at[j])
    if with_deg:
        pltpu.sync_copy(z1d, deg_s.at[pl.ds(s * (DEG_ROWS // NS),
                                            DEG_ROWS // NS)])
        for j in range(K // 16):
            degb[pl.ds(j * 16, 16)] = jnp.ones((16,), jnp.float32)

    @pl.when(s == 0)
    def _():
        pltpu.sync_copy(z2d.at[pl.ds(0, tail)], acc.at[pl.ds(NS * RPT, tail)])
    plsc.subcore_barrier()

    for j in range(3):
        pltpu.async_copy(table.at[sring.at[j]], rows.at[j], gsem)

    def step(i, carry):
        b = lax.rem(i, 4)
        pltpu.make_async_copy(table.at[sring.at[0]], rows.at[b], gsem).wait()

        @pl.when(i >= 1)
        def _():
            pltpu.make_async_copy(rows.at[0], acc.at[dring.at[0]],
                                  ssem).wait()
            if with_deg:
                pltpu.make_async_copy(degb, deg_s.at[dring.at[0]],
                                      dsem).wait()

        @pl.when((i >= 1) & (i + 3 < NCH))
        def _():
            pltpu.make_async_copy(src1d.at[pl.ds(ib, K)], sring.at[0],
                                  isem).wait()

        @pl.when((i >= 1) & (i + 2 < NCH))
        def _():
            pltpu.make_async_copy(dst1d.at[pl.ds(ib, K)], dring.at[0],
                                  vsem).wait()

        @pl.when(i + 3 < NCH)
        def _():
            pltpu.async_copy(table.at[sring.at[lax.rem(i + 3, 4)]],
                             rows.at[lax.rem(i + 3, 4)], gsem)
            pltpu.async_copy(dst1d.at[pl.ds(ib + (i + 3) * K, K)],
                             dring.at[lax.rem(i + 3, 4)], vsem)

        @pl.when(i + 4 < NCH)
        def _():
            pltpu.async_copy(src1d.at[pl.ds(ib + (i + 4) * K, K)],
                             sring.at[lax.rem(i, 4)], isem)

        pltpu.async_copy(rows.at[b], acc.at[dring.at[lax.rem(i, 4)]],
                         ssem, add=True)
        if with_deg:
            pltpu.async_copy(degb, deg_s.at[dring.at[lax.rem(i, 4)]],
                             dsem, add=True)
        return carry

    lax.fori_loop(0, NCH, step, 0)
    pltpu.make_async_copy(rows.at[0], acc.at[dring.at[0]], ssem).wait()
    if with_deg:
        pltpu.make_async_copy(degb, deg_s.at[dring.at[0]], dsem).wait()
    plsc.subcore_barrier()

    pltpu.sync_copy(acc.at[pl.ds(s * RPT, RPT)], out.at[c, pl.ds(s * RPT, RPT)])
    if with_deg:
        for j, sz in tuple((j, K) for j in range(7)) + ((7, RPT - 7 * K),):
            pltpu.sync_copy(deg_s.at[pl.ds(s * RPT + j * K, sz)],
                            degb.at[pl.ds(0, sz)])
            pltpu.sync_copy(degb.at[pl.ds(0, sz)],
                            degout.at[pl.ds(c * N_NODES + s * RPT + j * K,
                                            sz)])

    @pl.when(s == 0)
    def _():
        pltpu.sync_copy(acc.at[pl.ds(NS * RPT, tail)],
                        out.at[c, pl.ds(NS * RPT, tail)])
        if with_deg:
            pltpu.sync_copy(deg_s.at[pl.ds(NS * RPT, tail)],
                            degb.at[pl.ds(0, tail)])
            pltpu.sync_copy(degb.at[pl.ds(0, tail)],
                            degout.at[pl.ds(c * N_NODES + NS * RPT, tail)])


def _make_sc_agg(with_deg):
    mesh = plsc.VectorSubcoreMesh(core_axis_name="c", subcore_axis_name="s")
    common = [
        pltpu.VMEM_SHARED((ACC_ROWS, D), jnp.float32),
    ]
    bufs = [
        pltpu.VMEM((4, K), jnp.int32),
        pltpu.VMEM((4, K), jnp.int32),
        pltpu.VMEM((4, K, D), jnp.float32),
    ]
    sems = [pltpu.SemaphoreType.DMA] * 5
    if with_deg:
        out_type = (
            jax.ShapeDtypeStruct((NC, N_NODES, D), jnp.float32),
            jax.ShapeDtypeStruct((NC * N_NODES,), jnp.float32),
        )
        scratch = common + [pltpu.VMEM_SHARED((DEG_ROWS,), jnp.float32)] \
            + bufs + [pltpu.VMEM((K,), jnp.float32)] + sems
    else:
        out_type = jax.ShapeDtypeStruct((NC, N_NODES, D), jnp.float32)
        scratch = common + bufs + sems
    return pl.kernel(
        functools.partial(_sc_agg_body, with_deg),
        out_type=out_type,
        mesh=mesh,
        scratch_types=scratch,
    )


BR = 1000


def _mean_agg(sp_ref, degT_ref):
    ssum = sp_ref[0] + sp_ref[1]
    deg = jnp.sum(degT_ref[...], axis=1, keepdims=True)
    invd = 1.0 / jnp.clip(deg, 1.0, None)
    return ssum * invd


def _dense1_body(x_ref, sp_ref, degT_ref, Ws_ref, Wn_ref, b_ref, o_ref):
    agg = _mean_agg(sp_ref, degT_ref)
    h = jnp.dot(x_ref[...], Ws_ref[...], preferred_element_type=jnp.float32)
    h = h + jnp.dot(agg, Wn_ref[...], preferred_element_type=jnp.float32)
    h = h + b_ref[...][None, :]
    o_ref[...] = jnp.maximum(h, 0.0)


def _dense2_body(x_ref, sp_ref, degT_ref, Ws_ref, Wn_ref, b_ref, Wo_ref,
                 bo_ref, o_ref):
    agg = _mean_agg(sp_ref, degT_ref)
    h = jnp.dot(x_ref[...], Ws_ref[...], preferred_element_type=jnp.float32)
    h = h + jnp.dot(agg, Wn_ref[...], preferred_element_type=jnp.float32)
    h = jnp.maximum(h + b_ref[...][None, :], 0.0)
    o_ref[...] = (
        jnp.dot(h, Wo_ref[...], preferred_element_type=jnp.float32)
        + bo_ref[...][None, :]
    )


_W_SPEC = pl.BlockSpec((D, D), lambda i: (0, 0))
_B_SPEC = pl.BlockSpec((D,), lambda i: (0,))
_X_SPEC = pl.BlockSpec((BR, D), lambda i: (i, 0))
_SP_SPEC = pl.BlockSpec((NC, BR, D), lambda i: (0, i, 0))
_DEG_SPEC = pl.BlockSpec((BR, NC), lambda i: (i, 0))

_dense1 = pl.pallas_call(
    _dense1_body,
    grid=(N_NODES // BR,),
    in_specs=[_X_SPEC, _SP_SPEC, _DEG_SPEC, _W_SPEC, _W_SPEC, _B_SPEC],
    out_specs=_X_SPEC,
    out_shape=jax.ShapeDtypeStruct((N_NODES, D), jnp.float32),
)

_dense2 = pl.pallas_call(
    _dense2_body,
    grid=(N_NODES // BR,),
    in_specs=[_X_SPEC, _SP_SPEC, _DEG_SPEC, _W_SPEC, _W_SPEC, _B_SPEC,
              pl.BlockSpec((D, N_CLASSES), lambda i: (0, 0)),
              pl.BlockSpec((N_CLASSES,), lambda i: (0,))],
    out_specs=pl.BlockSpec((BR, N_CLASSES), lambda i: (i, 0)),
    out_shape=jax.ShapeDtypeStruct((N_NODES, N_CLASSES), jnp.float32),
)

_sc_agg_deg = _make_sc_agg(True)
_sc_agg = _make_sc_agg(False)


def kernel(features, edge_index, W_self1, W_neigh1, b1, W_self2, W_neigh2,
           b2, W_out, b_out):
    src1d = edge_index[0]
    dst1d = edge_index[1]
    z2d = jnp.zeros((RPT, D), jnp.float32)
    z1d = jnp.zeros((DEG_ROWS // NS,), jnp.float32)

    sp1, deg_flat = _sc_agg_deg(features, src1d, dst1d, z2d, z1d)
    degT = deg_flat.reshape(NC, N_NODES).T
    h1 = _dense1(features, sp1, degT, W_self1, W_neigh1, b1)
    sp2 = _sc_agg(h1, src1d, dst1d, z2d, z1d)
    return _dense2(h1, sp2, degT, W_self2, W_neigh2, b2, W_out, b_out)
